# baseline (speedup 1.0000x reference)
import functools

import jax
import jax.numpy as jnp
from jax import lax
from jax.experimental import pallas as pl
from jax.experimental.pallas import tpu as pltpu
from jax.experimental.pallas import tpu_sc as plsc

_EMB = 1024
_ROWS = 8192
_NC = 2
_NS = 16
_NW = _NC * _NS
_RPW = _ROWS // _NW
_CHUNK = 32
_NCHUNK = _RPW // _CHUNK


@functools.partial(
    pl.kernel,
    mesh=plsc.VectorSubcoreMesh(core_axis_name="c", subcore_axis_name="s"),
    out_type=jax.ShapeDtypeStruct((_ROWS, _EMB), jnp.float32),
)
def _sc_row_copy(table_hbm, out_hbm):
    wid = lax.axis_index("s") * _NC + lax.axis_index("c")
    base = wid * _RPW
    pltpu.sync_copy(table_hbm.at[pl.ds(base, _RPW)],
                    out_hbm.at[pl.ds(base, _RPW)])


def kernel(x, table):
    del x
    return _sc_row_copy(table)[None]


# speedup vs baseline: 24.0954x; 24.0954x over previous
import functools

import jax
import jax.numpy as jnp
from jax import lax
from jax.experimental import pallas as pl
from jax.experimental.pallas import tpu as pltpu
from jax.experimental.pallas import tpu_sc as plsc

_EMB = 1024
_ROWS = 8192
_NC = 2
_NS = 16
_NW = _NC * _NS
_RPW = _ROWS // _NW
_CHUNK = 32
_NCHUNK = _RPW // _CHUNK


@functools.partial(
    pl.kernel,
    mesh=plsc.VectorSubcoreMesh(core_axis_name="c", subcore_axis_name="s"),
    out_type=jax.ShapeDtypeStruct((_ROWS, _EMB), jnp.float32),
    scratch_types=[
        pltpu.VMEM((_CHUNK, _EMB), jnp.float32),
        pltpu.VMEM((_CHUNK, _EMB), jnp.float32),
        pltpu.SemaphoreType.DMA,
        pltpu.SemaphoreType.DMA,
        pltpu.SemaphoreType.DMA,
        pltpu.SemaphoreType.DMA,
    ],
)
def _sc_row_copy(table_hbm, out_hbm, buf0, buf1, is0, is1, os0, os1):
    wid = lax.axis_index("s") * _NC + lax.axis_index("c")
    base = wid * _RPW
    bufs = (buf0, buf1)
    isems = (is0, is1)
    osems = (os0, os1)
    reads = [None, None]
    writes = [None, None]
    reads[0] = pltpu.async_copy(table_hbm.at[pl.ds(base, _CHUNK)], buf0, is0)
    for i in range(_NCHUNK):
        b = i % 2
        nb = (i + 1) % 2
        if i + 1 < _NCHUNK:
            r_next = base + (i + 1) * _CHUNK
            if writes[nb] is not None:
                writes[nb].wait()
            reads[nb] = pltpu.async_copy(
                table_hbm.at[pl.ds(r_next, _CHUNK)], bufs[nb], isems[nb])
        reads[b].wait()
        writes[b] = pltpu.async_copy(
            bufs[b], out_hbm.at[pl.ds(base + i * _CHUNK, _CHUNK)], osems[b])
    writes[0].wait()
    writes[1].wait()


def kernel(x, table):
    del x
    return _sc_row_copy(table)[None]


# speedup vs baseline: 24.3475x; 1.0105x over previous
import functools

import jax
import jax.numpy as jnp
from jax import lax
from jax.experimental import pallas as pl
from jax.experimental.pallas import tpu as pltpu
from jax.experimental.pallas import tpu_sc as plsc

_EMB = 1024
_ROWS = 8192
_NC = 2
_NS = 16
_NW = _NC * _NS
_RPW = _ROWS // _NW
_CHUNK = 16
_NCHUNK = _RPW // _CHUNK
_NBUF = 4


@functools.partial(
    pl.kernel,
    mesh=plsc.VectorSubcoreMesh(core_axis_name="c", subcore_axis_name="s"),
    out_type=jax.ShapeDtypeStruct((_ROWS, _EMB), jnp.float32),
    scratch_types=(
        [pltpu.VMEM((_CHUNK, _EMB), jnp.float32)] * _NBUF
        + [pltpu.SemaphoreType.DMA] * (2 * _NBUF)
    ),
)
def _sc_row_copy(table_hbm, out_hbm, *refs):
    bufs = refs[:_NBUF]
    isems = refs[_NBUF:2 * _NBUF]
    osems = refs[2 * _NBUF:]
    wid = lax.axis_index("s") * _NC + lax.axis_index("c")
    base = wid * _RPW
    reads = [None] * _NBUF
    writes = [None] * _NBUF
    for i in range(_NBUF - 1):
        reads[i] = pltpu.async_copy(
            table_hbm.at[pl.ds(base + i * _CHUNK, _CHUNK)], bufs[i], isems[i])
    for i in range(_NCHUNK):
        b = i % _NBUF
        j = i + _NBUF - 1
        if j < _NCHUNK:
            jb = j % _NBUF
            if writes[jb] is not None:
                writes[jb].wait()
            reads[jb] = pltpu.async_copy(
                table_hbm.at[pl.ds(base + j * _CHUNK, _CHUNK)], bufs[jb],
                isems[jb])
        reads[b].wait()
        writes[b] = pltpu.async_copy(
            bufs[b], out_hbm.at[pl.ds(base + i * _CHUNK, _CHUNK)], osems[b])
    for b in range(_NBUF):
        if writes[b] is not None:
            writes[b].wait()


def kernel(x, table):
    del x
    return _sc_row_copy(table)[None]


# speedup vs baseline: 25.0958x; 1.0307x over previous
import functools

import jax
import jax.numpy as jnp
from jax import lax
from jax.experimental import pallas as pl
from jax.experimental.pallas import tpu as pltpu
from jax.experimental.pallas import tpu_sc as plsc

_EMB = 1024
_ROWS = 8192
_NC = 2
_NS = 16
_NW = _NC * _NS
_RPW = _ROWS // _NW
_SPM_ROWS = 64
_CHUNK = 16
_NCHUNK = (_RPW - _SPM_ROWS) // _CHUNK
_NBUF = 4


@functools.partial(
    pl.kernel,
    mesh=plsc.VectorSubcoreMesh(core_axis_name="c", subcore_axis_name="s"),
    out_type=jax.ShapeDtypeStruct((_ROWS, _EMB), jnp.float32),
    scratch_types=(
        [pltpu.VMEM((_CHUNK, _EMB), jnp.float32)] * _NBUF
        + [pltpu.SemaphoreType.DMA] * (2 * _NBUF)
        + [pltpu.VMEM_SHARED((_NS, _SPM_ROWS, _EMB), jnp.float32)]
        + [pltpu.SemaphoreType.DMA] * 2
    ),
)
def _sc_row_copy(table_hbm, out_hbm, *refs):
    bufs = refs[:_NBUF]
    isems = refs[_NBUF:2 * _NBUF]
    osems = refs[2 * _NBUF:3 * _NBUF]
    spm = refs[3 * _NBUF]
    spm_is, spm_os = refs[3 * _NBUF + 1], refs[3 * _NBUF + 2]
    cid = lax.axis_index("c")
    sid = lax.axis_index("s")
    wid = sid * _NC + cid
    base = wid * _RPW
    spm_base = base + _NCHUNK * _CHUNK
    spm_read = pltpu.async_copy(
        table_hbm.at[pl.ds(spm_base, _SPM_ROWS)], spm.at[sid], spm_is)
    reads = [None] * _NBUF
    writes = [None] * _NBUF
    spm_write = None
    for i in range(_NBUF - 1):
        reads[i] = pltpu.async_copy(
            table_hbm.at[pl.ds(base + i * _CHUNK, _CHUNK)], bufs[i], isems[i])
    for i in range(_NCHUNK):
        b = i % _NBUF
        j = i + _NBUF - 1
        if j < _NCHUNK:
            jb = j % _NBUF
            if writes[jb] is not None:
                writes[jb].wait()
            reads[jb] = pltpu.async_copy(
                table_hbm.at[pl.ds(base + j * _CHUNK, _CHUNK)], bufs[jb],
                isems[jb])
        reads[b].wait()
        writes[b] = pltpu.async_copy(
            bufs[b], out_hbm.at[pl.ds(base + i * _CHUNK, _CHUNK)], osems[b])
        if i == _NCHUNK // 2:
            spm_read.wait()
            spm_write = pltpu.async_copy(
                spm.at[sid], out_hbm.at[pl.ds(spm_base, _SPM_ROWS)], spm_os)
    for b in range(_NBUF):
        if writes[b] is not None:
            writes[b].wait()
    spm_write.wait()


def kernel(x, table):
    del x
    return _sc_row_copy(table)[None]
